# Initial kernel scaffold; baseline (speedup 1.0000x reference)
import jax
import jax.numpy as jnp
from jax.experimental import pallas as pl


def kernel(inputs, values, row_idx, col_idx):
    raise NotImplementedError("write your pallas kernel here")


# speedup vs baseline: 5.0040x; 5.0040x over previous
import functools

import jax
import jax.numpy as jnp
from jax import lax
from jax.experimental import pallas as pl
from jax.experimental.pallas import tpu as pltpu
from jax.experimental.pallas import tpu_sc as plsc

N = 4096
B = 256
NNZ = 167772

NBLK = 16
BLKC = N // NBLK
BLK_ELEMS = N * BLKC
SLICE = BLK_ELEMS // 16
SPAD = 1024
ZB = 8192

PER16 = ((NNZ + 16 * 128 - 1) // (16 * 128)) * 128
NNZ_PAD = 16 * PER16

_mesh = plsc.VectorSubcoreMesh(core_axis_name="c", subcore_axis_name="s")


@functools.partial(
    pl.kernel,
    out_type=jax.ShapeDtypeStruct((NBLK * BLK_ELEMS,), jnp.float32),
    mesh=_mesh,
    scratch_types=[
        pltpu.VMEM((NNZ_PAD // 16,), jnp.int32),
        pltpu.VMEM((NNZ_PAD // 16,), jnp.int32),
        pltpu.VMEM((NNZ_PAD // 16,), jnp.float32),
        pltpu.VMEM((128,), jnp.int32),
        pltpu.VMEM((ZB,), jnp.float32),
        pltpu.VMEM_SHARED((BLK_ELEMS + SPAD,), jnp.float32),
    ],
)
def _densify(vals_hbm, rows_hbm, cols_hbm, w_hbm,
             rows_v, cols_v, vals_v, idxbuf, zbuf, sblk):
    c = lax.axis_index("c")
    s = lax.axis_index("s")
    base = s * PER16
    pltpu.sync_copy(rows_hbm.at[pl.ds(base, PER16)], rows_v)
    pltpu.sync_copy(cols_hbm.at[pl.ds(base, PER16)], cols_v)
    pltpu.sync_copy(vals_hbm.at[pl.ds(base, PER16)], vals_v)

    zero16 = jnp.zeros((16,), jnp.float32)

    def _zb(i, carry):
        zbuf[pl.ds(i * 16, 16)] = zero16
        return carry

    lax.fori_loop(0, ZB // 16, _zb, 0)

    dump = BLK_ELEMS + s * 64

    for blk_i in range(NBLK // 2):
        blk = c * (NBLK // 2) + blk_i
        def _zs(i, carry):
            pltpu.sync_copy(zbuf, sblk.at[pl.ds(s * SLICE + i * ZB, ZB)])
            return carry

        lax.fori_loop(0, SLICE // ZB, _zs, 0)
        plsc.subcore_barrier()

        def _row(jj, carry):
            for kk in range(8):
                off = jj * 128 + kk * 16
                r = rows_v[pl.ds(off, 16)]
                cc = cols_v[pl.ds(off, 16)]
                valid = (cc >> 8) == blk
                idx = r * BLKC + (cc & (BLKC - 1))
                idxbuf[pl.ds(kk * 16, 16)] = jnp.where(valid, idx, dump)
            pltpu.sync_copy(vals_v.at[pl.ds(jj * 128, 128)],
                            sblk.at[idxbuf], add=True)
            return carry

        lax.fori_loop(0, PER16 // 128, _row, 0)
        plsc.subcore_barrier()

        pltpu.sync_copy(
            sblk.at[pl.ds(s * SLICE, SLICE)],
            w_hbm.at[pl.ds(blk * BLK_ELEMS + s * SLICE, SLICE)])


def _mm_body(x_ref, w_ref, o_ref):
    o_ref[...] = jnp.maximum(
        jnp.dot(x_ref[...], w_ref[0], preferred_element_type=jnp.float32), 0.0)


def _matmul_relu(x, w):
    return pl.pallas_call(
        _mm_body,
        grid=(NBLK,),
        in_specs=[
            pl.BlockSpec((B, N), lambda i: (0, 0)),
            pl.BlockSpec((1, N, BLKC), lambda i: (i, 0, 0)),
        ],
        out_specs=pl.BlockSpec((B, BLKC), lambda i: (0, i)),
        out_shape=jax.ShapeDtypeStruct((B, N), jnp.float32),
    )(x, w)


def kernel(inputs, values, row_idx, col_idx):
    pad = NNZ_PAD - NNZ
    vals_p = jnp.pad(values, (0, pad))
    rows_p = jnp.pad(row_idx, (0, pad))
    cols_p = jnp.pad(col_idx, (0, pad), constant_values=N)
    w_flat = _densify(vals_p, rows_p, cols_p)
    w = w_flat.reshape(NBLK, N, BLKC)
    return _matmul_relu(inputs, w)


# speedup vs baseline: 6.4496x; 1.2889x over previous
import functools

import jax
import jax.numpy as jnp
from jax import lax
from jax.experimental import pallas as pl
from jax.experimental.pallas import tpu as pltpu
from jax.experimental.pallas import tpu_sc as plsc

N = 4096
B = 256
NNZ = 167772

NBLK = 16
BLKC = N // NBLK
BLK_ELEMS = N * BLKC
SLICE = BLK_ELEMS // 16
SPAD = 1024
ZB = 32768

PER16 = ((NNZ + 16 * 128 - 1) // (16 * 128)) * 128
NNZ_PAD = 16 * PER16

_mesh = plsc.VectorSubcoreMesh(core_axis_name="c", subcore_axis_name="s")


@functools.partial(
    pl.kernel,
    out_type=jax.ShapeDtypeStruct((NBLK * BLK_ELEMS,), jnp.float32),
    mesh=_mesh,
    scratch_types=[
        pltpu.VMEM((NNZ_PAD // 16,), jnp.int32),
        pltpu.VMEM((NNZ_PAD // 16,), jnp.int32),
        pltpu.VMEM((NNZ_PAD // 16,), jnp.float32),
        pltpu.VMEM((128,), jnp.int32),
        pltpu.VMEM((ZB,), jnp.float32),
        pltpu.VMEM_SHARED((BLK_ELEMS + SPAD,), jnp.float32),
    ],
)
def _densify(vals_hbm, rows_hbm, cols_hbm, w_hbm,
             rows_v, cols_v, vals_v, idxbuf, zbuf, sblk):
    c = lax.axis_index("c")
    s = lax.axis_index("s")
    base = s * PER16
    pltpu.sync_copy(rows_hbm.at[pl.ds(base, PER16)], rows_v)
    pltpu.sync_copy(cols_hbm.at[pl.ds(base, PER16)], cols_v)
    pltpu.sync_copy(vals_hbm.at[pl.ds(base, PER16)], vals_v)

    zero16 = jnp.zeros((16,), jnp.float32)

    def _zb(i, carry):
        zbuf[pl.ds(i * 16, 16)] = zero16
        return carry

    lax.fori_loop(0, ZB // 16, _zb, 0)

    dump = BLK_ELEMS + s * 64

    nrows = PER16 // 128

    def _first_row(pred):
        def step(_, lohi):
            lo, hi = lohi
            mid = (lo + hi) >> 1
            p = pred(jnp.minimum(mid, nrows - 1)) | (lo >= hi)
            return (jnp.where(p, lo, mid + 1), jnp.where(p, mid, hi))

        lo, _ = lax.fori_loop(0, 7, step, (jnp.int32(0), jnp.int32(nrows)))
        return lo

    for blk_i in range(NBLK // 2):
        blk = c * (NBLK // 2) + blk_i
        def _zs(i, carry):
            pltpu.sync_copy(zbuf, sblk.at[pl.ds(s * SLICE + i * ZB, ZB)])
            return carry

        lax.fori_loop(0, SLICE // ZB, _zs, 0)
        plsc.subcore_barrier()

        r_lo = _first_row(
            lambda r: cols_v[pl.ds(r * 128 + 112, 16)][15] >= blk * BLKC)
        r_hi = _first_row(
            lambda r: cols_v[pl.ds(r * 128, 16)][0] >= (blk + 1) * BLKC)

        def _row(jj, carry):
            for kk in range(8):
                off = jj * 128 + kk * 16
                r = rows_v[pl.ds(off, 16)]
                cc = cols_v[pl.ds(off, 16)]
                valid = (cc >> 8) == blk
                idx = r * BLKC + (cc & (BLKC - 1))
                idxbuf[pl.ds(kk * 16, 16)] = jnp.where(valid, idx, dump)
            pltpu.sync_copy(vals_v.at[pl.ds(jj * 128, 128)],
                            sblk.at[idxbuf], add=True)
            return carry

        lax.fori_loop(r_lo, r_hi, _row, 0)
        plsc.subcore_barrier()

        pltpu.sync_copy(
            sblk.at[pl.ds(s * SLICE, SLICE)],
            w_hbm.at[pl.ds(blk * BLK_ELEMS + s * SLICE, SLICE)])


def _mm_body(x_ref, w_ref, o_ref):
    o_ref[...] = jnp.maximum(
        jnp.dot(x_ref[...], w_ref[0], preferred_element_type=jnp.float32), 0.0)


def _matmul_relu(x, w):
    return pl.pallas_call(
        _mm_body,
        grid=(NBLK,),
        in_specs=[
            pl.BlockSpec((B, N), lambda i: (0, 0)),
            pl.BlockSpec((1, N, BLKC), lambda i: (i, 0, 0)),
        ],
        out_specs=pl.BlockSpec((B, BLKC), lambda i: (0, i)),
        out_shape=jax.ShapeDtypeStruct((B, N), jnp.float32),
    )(x, w)


def kernel(inputs, values, row_idx, col_idx):
    pad = NNZ_PAD - NNZ
    vals_p = jnp.pad(values, (0, pad))
    rows_p = jnp.pad(row_idx, (0, pad))
    cols_p = jnp.pad(col_idx, (0, pad), constant_values=N)
    w_flat = _densify(vals_p, rows_p, cols_p)
    w = w_flat.reshape(NBLK, N, BLKC)
    return _matmul_relu(inputs, w)


# speedup vs baseline: 8.7177x; 1.3517x over previous
import functools

import jax
import jax.numpy as jnp
from jax import lax
from jax.experimental import pallas as pl
from jax.experimental.pallas import tpu as pltpu
from jax.experimental.pallas import tpu_sc as plsc

N = 4096
B = 256
NNZ = 167772

NBLK = 16
BLKC = N // NBLK
BLK_ELEMS = N * BLKC
SLICE = BLK_ELEMS // 16
SPAD = 1024
ZB = 16384

PER16 = ((NNZ + 16 * 128 - 1) // (16 * 128)) * 128
NNZ_PAD = 16 * PER16

_mesh = plsc.VectorSubcoreMesh(core_axis_name="c", subcore_axis_name="s")


@functools.partial(
    pl.kernel,
    out_type=jax.ShapeDtypeStruct((NBLK * BLK_ELEMS,), jnp.float32),
    mesh=_mesh,
    scratch_types=[
        pltpu.VMEM((NNZ_PAD // 16,), jnp.int32),
        pltpu.VMEM((NNZ_PAD // 16,), jnp.int32),
        pltpu.VMEM((NNZ_PAD // 16,), jnp.float32),
        pltpu.VMEM((NNZ_PAD // 16 // 128, 128), jnp.int32),
        pltpu.VMEM((128,), jnp.int32),
        pltpu.VMEM((ZB,), jnp.float32),
        pltpu.VMEM_SHARED((BLK_ELEMS + SPAD,), jnp.float32),
        pltpu.SemaphoreType.DMA,
        pltpu.SemaphoreType.DMA,
    ],
)
def _densify(vals_hbm, rows_hbm, cols_hbm, w_hbm,
             rows_v, cols_v, vals_v, idx_full, bbuf, zbuf, sblk, sem, semz):
    c = lax.axis_index("c")
    s = lax.axis_index("s")
    base = s * PER16
    a1 = pltpu.async_copy(rows_hbm.at[pl.ds(base, PER16)], rows_v, sem)
    a2 = pltpu.async_copy(cols_hbm.at[pl.ds(base, PER16)], cols_v, sem)
    a3 = pltpu.async_copy(vals_hbm.at[pl.ds(base, PER16)], vals_v, sem)
    a1.wait()
    a2.wait()
    a3.wait()

    zero16 = jnp.zeros((16,), jnp.float32)

    def _zb(i, carry):
        zbuf[pl.ds(i * 16, 16)] = zero16
        return carry

    lax.fori_loop(0, ZB // 16, _zb, 0)

    nrows = PER16 // 128

    def _pre(r, carry):
        for kk in range(8):
            off = r * 128 + kk * 16
            rr = rows_v[pl.ds(off, 16)]
            cc = cols_v[pl.ds(off, 16)]
            idx_full[r, pl.ds(kk * 16, 16)] = rr * BLKC + (cc & (BLKC - 1))
        return carry

    lax.fori_loop(0, nrows, _pre, 0)

    dump = BLK_ELEMS + s * 64

    def _first_row(pred):
        def step(_, lohi):
            lo, hi = lohi
            mid = (lo + hi) >> 1
            p = pred(jnp.minimum(mid, nrows - 1)) | (lo >= hi)
            return (jnp.where(p, lo, mid + 1), jnp.where(p, mid, hi))

        lo, _ = lax.fori_loop(0, 7, step, (jnp.int32(0), jnp.int32(nrows)))
        return lo

    for blk_i in range(NBLK // 2):
        blk = c * (NBLK // 2) + blk_i
        def _zf(i, carry):
            pltpu.async_copy(zbuf, sblk.at[pl.ds(s * SLICE + i * ZB, ZB)], semz)
            return carry

        def _zd(i, carry):
            pltpu.make_async_copy(
                zbuf, sblk.at[pl.ds(s * SLICE + i * ZB, ZB)], semz).wait()
            return carry

        lax.fori_loop(0, SLICE // ZB, _zf, 0)
        lax.fori_loop(0, SLICE // ZB, _zd, 0)
        plsc.subcore_barrier()

        a = _first_row(lambda r: cols_v[pl.ds(r * 128, 16)][0] >= blk * BLKC)
        b2 = _first_row(
            lambda r: cols_v[pl.ds(r * 128, 16)][0] >= (blk + 1) * BLKC)
        interior_end = jnp.maximum(b2 - 1, a)

        def _if(r, carry):
            pltpu.async_copy(vals_v.at[pl.ds(r * 128, 128)],
                             sblk.at[idx_full.at[r]], sem, add=True)
            return carry

        def _id(r, carry):
            pltpu.make_async_copy(vals_v.at[pl.ds(r * 128, 128)],
                                  sblk.at[idx_full.at[r]], sem).wait()
            return carry

        lax.fori_loop(a, interior_end, _if, 0)

        def _brow(r):
            for kk in range(8):
                off = r * 128 + kk * 16
                cc = cols_v[pl.ds(off, 16)]
                valid = (cc >> 8) == blk
                iv = idx_full[r, pl.ds(kk * 16, 16)]
                bbuf[pl.ds(kk * 16, 16)] = jnp.where(valid, iv, dump)
            pltpu.sync_copy(vals_v.at[pl.ds(r * 128, 128)],
                            sblk.at[bbuf], add=True)

        @pl.when(a > 0)
        def _():
            _brow(a - 1)

        @pl.when(b2 > a)
        def _():
            _brow(b2 - 1)

        lax.fori_loop(a, interior_end, _id, 0)
        plsc.subcore_barrier()

        pltpu.sync_copy(
            sblk.at[pl.ds(s * SLICE, SLICE)],
            w_hbm.at[pl.ds(blk * BLK_ELEMS + s * SLICE, SLICE)])


def _mm_body(x_ref, w_ref, o_ref):
    o_ref[...] = jnp.maximum(
        jnp.dot(x_ref[...], w_ref[0], preferred_element_type=jnp.float32), 0.0)


def _matmul_relu(x, w):
    return pl.pallas_call(
        _mm_body,
        grid=(NBLK,),
        in_specs=[
            pl.BlockSpec((B, N), lambda i: (0, 0)),
            pl.BlockSpec((1, N, BLKC), lambda i: (i, 0, 0)),
        ],
        out_specs=pl.BlockSpec((B, BLKC), lambda i: (0, i)),
        out_shape=jax.ShapeDtypeStruct((B, N), jnp.float32),
    )(x, w)


def kernel(inputs, values, row_idx, col_idx):
    pad = NNZ_PAD - NNZ
    vals_p = jnp.pad(values, (0, pad))
    rows_p = jnp.pad(row_idx, (0, pad))
    cols_p = jnp.pad(col_idx, (0, pad), constant_values=N)
    w_flat = _densify(vals_p, rows_p, cols_p)
    w = w_flat.reshape(NBLK, N, BLKC)
    return _matmul_relu(inputs, w)


# speedup vs baseline: 9.8668x; 1.1318x over previous
import functools

import jax
import jax.numpy as jnp
from jax import lax
from jax.experimental import pallas as pl
from jax.experimental.pallas import tpu as pltpu
from jax.experimental.pallas import tpu_sc as plsc

N = 4096
B = 256
NNZ = 167772

NBLK = 16
BLKC = N // NBLK
BLK_ELEMS = N * BLKC
SLICE = BLK_ELEMS // 16
SPAD = 1024
ZB = 16384

PER16 = ((NNZ + 16 * 128 - 1) // (16 * 128)) * 128
NNZ_PAD = 16 * PER16
NROWS = PER16 // 128

HALVES = 2
NBLK_CALL = NBLK // HALVES

_mesh = plsc.VectorSubcoreMesh(core_axis_name="c", subcore_axis_name="s")


def _make_densify(h):
    def _densify(vals_hbm, rows_hbm, cols_hbm, w_hbm,
                 rows_v, cols_v, vals_v, idx_full, bbuf, zbuf, sblk,
                 sem, semz):
        c = lax.axis_index("c")
        s = lax.axis_index("s")
        base = s * PER16
        a1 = pltpu.async_copy(rows_hbm.at[pl.ds(base, PER16)], rows_v, sem)
        a2 = pltpu.async_copy(cols_hbm.at[pl.ds(base, PER16)], cols_v, sem)
        a3 = pltpu.async_copy(vals_hbm.at[pl.ds(base, PER16)], vals_v, sem)
        a1.wait()
        a2.wait()
        a3.wait()

        zero16 = jnp.zeros((16,), jnp.float32)

        def _zb(i, carry):
            zbuf[pl.ds(i * 16, 16)] = zero16
            return carry

        lax.fori_loop(0, ZB // 16, _zb, 0)

        def _pre(r, carry):
            for kk in range(8):
                off = r * 128 + kk * 16
                rr = rows_v[pl.ds(off, 16)]
                cc = cols_v[pl.ds(off, 16)]
                idx_full[r, pl.ds(kk * 16, 16)] = rr * BLKC + (cc & (BLKC - 1))
            return carry

        lax.fori_loop(0, NROWS, _pre, 0)

        dump = BLK_ELEMS + s * 64

        def _first_row(pred):
            def step(_, lohi):
                lo, hi = lohi
                mid = (lo + hi) >> 1
                p = pred(jnp.minimum(mid, NROWS - 1)) | (lo >= hi)
                return (jnp.where(p, lo, mid + 1), jnp.where(p, mid, hi))

            lo, _ = lax.fori_loop(0, 7, step, (jnp.int32(0), jnp.int32(NROWS)))
            return lo

        for blk_i in range(NBLK_CALL // 2):
            lb = c * (NBLK_CALL // 2) + blk_i
            blk = h * NBLK_CALL + lb

            def _zf(i, carry):
                pltpu.async_copy(zbuf, sblk.at[pl.ds(s * SLICE + i * ZB, ZB)],
                                 semz)
                return carry

            def _zd(i, carry):
                pltpu.make_async_copy(
                    zbuf, sblk.at[pl.ds(s * SLICE + i * ZB, ZB)], semz).wait()
                return carry

            lax.fori_loop(0, SLICE // ZB, _zf, 0)
            lax.fori_loop(0, SLICE // ZB, _zd, 0)
            plsc.subcore_barrier()

            a = _first_row(
                lambda r: cols_v[pl.ds(r * 128, 16)][0] >= blk * BLKC)
            b2 = _first_row(
                lambda r: cols_v[pl.ds(r * 128, 16)][0] >= (blk + 1) * BLKC)
            interior_end = jnp.maximum(b2 - 1, a)

            def _if(r, carry):
                pltpu.async_copy(vals_v.at[pl.ds(r * 128, 128)],
                                 sblk.at[idx_full.at[r]], sem, add=True)
                return carry

            def _id(r, carry):
                pltpu.make_async_copy(vals_v.at[pl.ds(r * 128, 128)],
                                      sblk.at[idx_full.at[r]], sem).wait()
                return carry

            lax.fori_loop(a, interior_end, _if, 0)

            def _brow(r):
                for kk in range(8):
                    off = r * 128 + kk * 16
                    cc = cols_v[pl.ds(off, 16)]
                    valid = (cc >> 8) == blk
                    iv = idx_full[r, pl.ds(kk * 16, 16)]
                    bbuf[pl.ds(kk * 16, 16)] = jnp.where(valid, iv, dump)
                pltpu.sync_copy(vals_v.at[pl.ds(r * 128, 128)],
                                sblk.at[bbuf], add=True)

            @pl.when(a > 0)
            def _():
                _brow(a - 1)

            @pl.when(b2 > a)
            def _():
                _brow(b2 - 1)

            lax.fori_loop(a, interior_end, _id, 0)
            plsc.subcore_barrier()

            pltpu.sync_copy(
                sblk.at[pl.ds(s * SLICE, SLICE)],
                w_hbm.at[pl.ds(lb * BLK_ELEMS + s * SLICE, SLICE)])

    return functools.partial(
        pl.kernel,
        out_type=jax.ShapeDtypeStruct((NBLK_CALL * BLK_ELEMS,), jnp.float32),
        mesh=_mesh,
        scratch_types=[
            pltpu.VMEM((PER16,), jnp.int32),
            pltpu.VMEM((PER16,), jnp.int32),
            pltpu.VMEM((PER16,), jnp.float32),
            pltpu.VMEM((NROWS, 128), jnp.int32),
            pltpu.VMEM((128,), jnp.int32),
            pltpu.VMEM((ZB,), jnp.float32),
            pltpu.VMEM_SHARED((BLK_ELEMS + SPAD,), jnp.float32),
            pltpu.SemaphoreType.DMA,
            pltpu.SemaphoreType.DMA,
        ],
    )(_densify)


_densify_halves = tuple(_make_densify(h) for h in range(HALVES))


def _mm_body(x_ref, w_ref, o_ref):
    o_ref[...] = jnp.maximum(
        jnp.dot(x_ref[...], w_ref[0], preferred_element_type=jnp.float32), 0.0)


def _matmul_relu(x, w):
    nb = w.shape[0]
    return pl.pallas_call(
        _mm_body,
        grid=(nb,),
        in_specs=[
            pl.BlockSpec((B, N), lambda i: (0, 0)),
            pl.BlockSpec((1, N, BLKC), lambda i: (i, 0, 0)),
        ],
        out_specs=pl.BlockSpec((B, BLKC), lambda i: (0, i)),
        out_shape=jax.ShapeDtypeStruct((B, nb * BLKC), jnp.float32),
    )(x, w)


def kernel(inputs, values, row_idx, col_idx):
    pad = NNZ_PAD - NNZ
    vals_p = jnp.pad(values, (0, pad))
    rows_p = jnp.pad(row_idx, (0, pad))
    cols_p = jnp.pad(col_idx, (0, pad), constant_values=N)
    outs = []
    for h in range(HALVES):
        w_flat = _densify_halves[h](vals_p, rows_p, cols_p)
        outs.append(_matmul_relu(inputs, w_flat.reshape(NBLK_CALL, N, BLKC)))
    return jnp.concatenate(outs, axis=1)


# speedup vs baseline: 12.0908x; 1.2254x over previous
import functools

import jax
import jax.numpy as jnp
from jax import lax
from jax.experimental import pallas as pl
from jax.experimental.pallas import tpu as pltpu
from jax.experimental.pallas import tpu_sc as plsc

N = 4096
B = 256
NNZ = 167772

NBLK = 16
BLKC = N // NBLK
BLK_ELEMS = N * BLKC
SLICE = BLK_ELEMS // 16
SPAD = 1024
ZB = 16384

PER16 = ((NNZ + 16 * 128 - 1) // (16 * 128)) * 128
NNZ_PAD = 16 * PER16
NROWS = PER16 // 128

HALVES = 2
NBLK_CALL = NBLK // HALVES

_mesh = plsc.VectorSubcoreMesh(core_axis_name="c", subcore_axis_name="s")


def _make_densify(h):
    def _densify(vals_hbm, rows_hbm, cols_hbm, w_hbm,
                 rows_v, cols_v, vals_v, idx_full, bbuf, zbuf, sblk,
                 sem, semz):
        c = lax.axis_index("c")
        s = lax.axis_index("s")
        base = s * PER16
        a1 = pltpu.async_copy(rows_hbm.at[pl.ds(base, PER16)], rows_v, sem)
        a2 = pltpu.async_copy(cols_hbm.at[pl.ds(base, PER16)], cols_v, sem)
        a3 = pltpu.async_copy(vals_hbm.at[pl.ds(base, PER16)], vals_v, sem)
        a1.wait()
        a2.wait()
        a3.wait()

        zero16 = jnp.zeros((16,), jnp.float32)

        def _zb(i, carry):
            zbuf[pl.ds(i * 16, 16)] = zero16
            return carry

        lax.fori_loop(0, ZB // 16, _zb, 0)

        def _pre(r, carry):
            for kk in range(8):
                off = r * 128 + kk * 16
                rr = rows_v[pl.ds(off, 16)]
                cc = cols_v[pl.ds(off, 16)]
                idx_full[r, pl.ds(kk * 16, 16)] = rr * BLKC + (cc & (BLKC - 1))
            return carry

        lax.fori_loop(0, NROWS, _pre, 0)

        dump = BLK_ELEMS + s * 64

        def _first_row(pred):
            def step(_, lohi):
                lo, hi = lohi
                mid = (lo + hi) >> 1
                p = pred(jnp.minimum(mid, NROWS - 1)) | (lo >= hi)
                return (jnp.where(p, lo, mid + 1), jnp.where(p, mid, hi))

            lo, _ = lax.fori_loop(0, 7, step, (jnp.int32(0), jnp.int32(NROWS)))
            return lo

        for blk_i in range(NBLK_CALL // 2):
            lb = c * (NBLK_CALL // 2) + blk_i
            blk = h * NBLK_CALL + lb

            def _zf(i, carry):
                pltpu.async_copy(zbuf, sblk.at[pl.ds(s * SLICE + i * ZB, ZB)],
                                 semz)
                return carry

            def _zd(i, carry):
                pltpu.make_async_copy(
                    zbuf, sblk.at[pl.ds(s * SLICE + i * ZB, ZB)], semz).wait()
                return carry

            lax.fori_loop(0, SLICE // ZB, _zf, 0)
            lax.fori_loop(0, SLICE // ZB, _zd, 0)
            plsc.subcore_barrier()

            a = _first_row(
                lambda r: cols_v[pl.ds(r * 128, 16)][0] >= blk * BLKC)
            b2 = _first_row(
                lambda r: cols_v[pl.ds(r * 128, 16)][0] >= (blk + 1) * BLKC)
            interior_end = jnp.maximum(b2 - 1, a)

            def _if(r, carry):
                pltpu.async_copy(vals_v.at[pl.ds(r * 128, 128)],
                                 sblk.at[idx_full.at[r]], sem, add=True)
                return carry

            def _id(r, carry):
                pltpu.make_async_copy(vals_v.at[pl.ds(r * 128, 128)],
                                      sblk.at[idx_full.at[r]], sem).wait()
                return carry

            lax.fori_loop(a, interior_end, _if, 0)

            def _brow(r):
                for kk in range(8):
                    off = r * 128 + kk * 16
                    cc = cols_v[pl.ds(off, 16)]
                    valid = (cc >> 8) == blk
                    iv = idx_full[r, pl.ds(kk * 16, 16)]
                    bbuf[pl.ds(kk * 16, 16)] = jnp.where(valid, iv, dump)
                pltpu.sync_copy(vals_v.at[pl.ds(r * 128, 128)],
                                sblk.at[bbuf], add=True)

            @pl.when(a > 0)
            def _():
                _brow(a - 1)

            @pl.when(b2 > a)
            def _():
                _brow(b2 - 1)

            lax.fori_loop(a, interior_end, _id, 0)
            plsc.subcore_barrier()

            pltpu.sync_copy(
                sblk.at[pl.ds(s * SLICE, SLICE)],
                w_hbm.at[pl.ds(lb * BLK_ELEMS + s * SLICE, SLICE)])

    return functools.partial(
        pl.kernel,
        out_type=jax.ShapeDtypeStruct((NBLK_CALL * BLK_ELEMS,), jnp.float32),
        mesh=_mesh,
        scratch_types=[
            pltpu.VMEM((PER16,), jnp.int32),
            pltpu.VMEM((PER16,), jnp.int32),
            pltpu.VMEM((PER16,), jnp.float32),
            pltpu.VMEM((NROWS, 128), jnp.int32),
            pltpu.VMEM((128,), jnp.int32),
            pltpu.VMEM((ZB,), jnp.float32),
            pltpu.VMEM_SHARED((BLK_ELEMS + SPAD,), jnp.float32),
            pltpu.SemaphoreType.DMA,
            pltpu.SemaphoreType.DMA,
        ],
    )(_densify)


_densify_halves = tuple(_make_densify(h) for h in range(HALVES))


def _mm_body(x_ref, w_ref, o_ref):
    w = w_ref[...].reshape(N, BLKC)
    o_ref[...] = jnp.maximum(
        jnp.dot(x_ref[...], w, preferred_element_type=jnp.float32), 0.0)


def _matmul_relu(x, w_flat):
    nb = w_flat.shape[0] // BLK_ELEMS
    return pl.pallas_call(
        _mm_body,
        grid=(nb,),
        in_specs=[
            pl.BlockSpec((B, N), lambda i: (0, 0)),
            pl.BlockSpec((BLK_ELEMS,), lambda i: (i,)),
        ],
        out_specs=pl.BlockSpec((B, BLKC), lambda i: (0, i)),
        out_shape=jax.ShapeDtypeStruct((B, nb * BLKC), jnp.float32),
    )(x, w_flat)


def kernel(inputs, values, row_idx, col_idx):
    pad = NNZ_PAD - NNZ
    vals_p = jnp.pad(values, (0, pad))
    rows_p = jnp.pad(row_idx, (0, pad))
    cols_p = jnp.pad(col_idx, (0, pad), constant_values=N)
    outs = []
    for h in range(HALVES):
        w_flat = _densify_halves[h](vals_p, rows_p, cols_p)
        outs.append(_matmul_relu(inputs, w_flat))
    return jnp.concatenate(outs, axis=1)


# speedup vs baseline: 12.7661x; 1.0559x over previous
import functools

import jax
import jax.numpy as jnp
from jax import lax
from jax.experimental import pallas as pl
from jax.experimental.pallas import tpu as pltpu
from jax.experimental.pallas import tpu_sc as plsc

N = 4096
B = 256
NNZ = 167772

BLKC = 128
NBLK = N // BLKC
BLK_ELEMS = N * BLKC
SLICE = BLK_ELEMS // 16
SPAD = 1024
ZB = 16384

PER16 = ((NNZ + 16 * 128 - 1) // (16 * 128)) * 128
NNZ_PAD = 16 * PER16
NROWS = PER16 // 128

HALVES = 2
NBLK_CALL = NBLK // HALVES
BPC = NBLK_CALL // 2

_mesh = plsc.VectorSubcoreMesh(core_axis_name="c", subcore_axis_name="s")


def _make_densify(h):
    def _densify(vals_hbm, rows_hbm, cols_hbm, w_hbm,
                 rows_v, cols_v, vals_v, idx_full, bbuf, zbuf, sblk_a, sblk_b,
                 sem, semz, semc):
        c = lax.axis_index("c")
        s = lax.axis_index("s")
        base = s * PER16
        a1 = pltpu.async_copy(rows_hbm.at[pl.ds(base, PER16)], rows_v, sem)
        a2 = pltpu.async_copy(cols_hbm.at[pl.ds(base, PER16)], cols_v, sem)
        a3 = pltpu.async_copy(vals_hbm.at[pl.ds(base, PER16)], vals_v, sem)
        a1.wait()
        a2.wait()
        a3.wait()

        zero16 = jnp.zeros((16,), jnp.float32)

        def _zb(i, carry):
            zbuf[pl.ds(i * 16, 16)] = zero16
            return carry

        lax.fori_loop(0, ZB // 16, _zb, 0)

        def _pre(r, carry):
            for kk in range(8):
                off = r * 128 + kk * 16
                rr = rows_v[pl.ds(off, 16)]
                cc = cols_v[pl.ds(off, 16)]
                idx_full[r, pl.ds(kk * 16, 16)] = rr * BLKC + (cc & (BLKC - 1))
            return carry

        lax.fori_loop(0, NROWS, _pre, 0)

        dump = BLK_ELEMS + s * 64

        def _first_row(pred):
            def step(_, lohi):
                lo, hi = lohi
                mid = (lo + hi) >> 1
                p = pred(jnp.minimum(mid, NROWS - 1)) | (lo >= hi)
                return (jnp.where(p, lo, mid + 1), jnp.where(p, mid, hi))

            lo, _ = lax.fori_loop(0, 7, step,
                                  (jnp.int32(0), jnp.int32(NROWS)))
            return lo

        def _zero_fire(buf, i, sem_):
            pltpu.async_copy(zbuf, buf.at[pl.ds(s * SLICE + i * ZB, ZB)], sem_)

        def _zero_drain(buf, i, sem_):
            pltpu.make_async_copy(
                zbuf, buf.at[pl.ds(s * SLICE + i * ZB, ZB)], sem_).wait()

        def _scatter(buf, blk):
            a = _first_row(
                lambda r: cols_v[pl.ds(r * 128, 16)][0] >= blk * BLKC)
            b2 = _first_row(
                lambda r: cols_v[pl.ds(r * 128, 16)][0] >= (blk + 1) * BLKC)
            interior_end = jnp.maximum(b2 - 1, a)

            def _if(r, carry):
                pltpu.async_copy(vals_v.at[pl.ds(r * 128, 128)],
                                 buf.at[idx_full.at[r]], sem, add=True)
                return carry

            def _id(r, carry):
                pltpu.make_async_copy(vals_v.at[pl.ds(r * 128, 128)],
                                      buf.at[idx_full.at[r]], sem).wait()
                return carry

            lax.fori_loop(a, interior_end, _if, 0)

            def _brow(r):
                for kk in range(8):
                    off = r * 128 + kk * 16
                    cc = cols_v[pl.ds(off, 16)]
                    valid = (cc >> 7) == blk
                    iv = idx_full[r, pl.ds(kk * 16, 16)]
                    bbuf[pl.ds(kk * 16, 16)] = jnp.where(valid, iv, dump)
                pltpu.sync_copy(vals_v.at[pl.ds(r * 128, 128)],
                                buf.at[bbuf], add=True)

            @pl.when(a > 0)
            def _():
                _brow(a - 1)

            @pl.when(b2 > a)
            def _():
                _brow(b2 - 1)

            lax.fori_loop(a, interior_end, _id, 0)

        def _out_slices(j):
            lb = c * BPC + j
            blk = h * NBLK_CALL + lb
            return blk, lb * BLK_ELEMS + s * SLICE

        for i in range(SLICE // ZB):
            _zero_fire(sblk_a, i, semz)
        for i in range(SLICE // ZB):
            _zero_drain(sblk_a, i, semz)
        plsc.subcore_barrier()

        for j in range(BPC):
            cur, nxt = (sblk_a, sblk_b) if j % 2 == 0 else (sblk_b, sblk_a)
            blk, w_off = _out_slices(j)
            if j > 0:
                _, w_off_prev = _out_slices(j - 1)
                pltpu.async_copy(nxt.at[pl.ds(s * SLICE, SLICE)],
                                 w_hbm.at[pl.ds(w_off_prev, SLICE)], semc)
            _scatter(cur, blk)
            if j > 0:
                pltpu.make_async_copy(nxt.at[pl.ds(s * SLICE, SLICE)],
                                      w_hbm.at[pl.ds(w_off_prev, SLICE)],
                                      semc).wait()
            if j < BPC - 1:
                for i in range(SLICE // ZB):
                    _zero_fire(nxt, i, semz)
                for i in range(SLICE // ZB):
                    _zero_drain(nxt, i, semz)
            plsc.subcore_barrier()

        last = sblk_a if (BPC - 1) % 2 == 0 else sblk_b
        _, w_off_last = _out_slices(BPC - 1)
        pltpu.sync_copy(last.at[pl.ds(s * SLICE, SLICE)],
                        w_hbm.at[pl.ds(w_off_last, SLICE)])

    return functools.partial(
        pl.kernel,
        out_type=jax.ShapeDtypeStruct((NBLK_CALL * BLK_ELEMS,), jnp.float32),
        mesh=_mesh,
        scratch_types=[
            pltpu.VMEM((PER16,), jnp.int32),
            pltpu.VMEM((PER16,), jnp.int32),
            pltpu.VMEM((PER16,), jnp.float32),
            pltpu.VMEM((NROWS, 128), jnp.int32),
            pltpu.VMEM((128,), jnp.int32),
            pltpu.VMEM((ZB,), jnp.float32),
            pltpu.VMEM_SHARED((BLK_ELEMS + SPAD,), jnp.float32),
            pltpu.VMEM_SHARED((BLK_ELEMS + SPAD,), jnp.float32),
            pltpu.SemaphoreType.DMA,
            pltpu.SemaphoreType.DMA,
            pltpu.SemaphoreType.DMA,
        ],
    )(_densify)


_densify_halves = tuple(_make_densify(h) for h in range(HALVES))


def _mm_body(x_ref, w_ref, o_ref):
    w = w_ref[...].reshape(N, BLKC)
    o_ref[...] = jnp.maximum(
        jnp.dot(x_ref[...], w, preferred_element_type=jnp.float32), 0.0)


def _matmul_relu(x, w_flat):
    nb = w_flat.shape[0] // BLK_ELEMS
    return pl.pallas_call(
        _mm_body,
        grid=(nb,),
        in_specs=[
            pl.BlockSpec((B, N), lambda i: (0, 0)),
            pl.BlockSpec((BLK_ELEMS,), lambda i: (i,)),
        ],
        out_specs=pl.BlockSpec((B, BLKC), lambda i: (0, i)),
        out_shape=jax.ShapeDtypeStruct((B, nb * BLKC), jnp.float32),
    )(x, w_flat)


def kernel(inputs, values, row_idx, col_idx):
    pad = NNZ_PAD - NNZ
    vals_p = jnp.pad(values, (0, pad))
    rows_p = jnp.pad(row_idx, (0, pad))
    cols_p = jnp.pad(col_idx, (0, pad), constant_values=N)
    outs = []
    for h in range(HALVES):
        w_flat = _densify_halves[h](vals_p, rows_p, cols_p)
        outs.append(_matmul_relu(inputs, w_flat))
    return jnp.concatenate(outs, axis=1)
